# Initial kernel scaffold; baseline (speedup 1.0000x reference)
import jax
import jax.numpy as jnp
from jax.experimental import pallas as pl


def kernel(x, pos, batch, params):
    raise NotImplementedError("write your pallas kernel here")


# speedup vs baseline: 1.0198x; 1.0198x over previous
import jax
import jax.numpy as jnp
import numpy as np
from functools import partial
from jax.experimental import pallas as pl
from jax.experimental.pallas import tpu as pltpu


def _fps_jnp(pos, S):
    P = pos.shape[0]
    def body(i, state):
        dist, idx = state
        last = idx[i - 1]
        d = jnp.sum((pos - pos[last]) ** 2, axis=-1)
        dist = jnp.minimum(dist, d)
        idx = idx.at[i].set(jnp.argmax(dist).astype(jnp.int32))
        return dist, idx
    dist0 = jnp.full((P,), jnp.inf, dtype=pos.dtype)
    idx0 = jnp.zeros((S,), dtype=jnp.int32)
    _, idx = jax.lax.fori_loop(1, S, body, (dist0, idx0))
    return idx


def _mlp_bn_jnp(h, mask, layers):
    for (W, b, g, beta) in layers:
        h = h @ W + b
        if mask is None:
            mu = jnp.mean(h, axis=0)
            var = jnp.mean((h - mu) ** 2, axis=0)
        else:
            m = mask[:, None].astype(h.dtype)
            cnt = jnp.maximum(jnp.sum(m), 1.0)
            mu = jnp.sum(h * m, axis=0) / cnt
            var = jnp.sum(((h - mu) ** 2) * m, axis=0) / cnt
        h = (h - mu) / jnp.sqrt(var + 1e-5) * g + beta
        h = jax.nn.relu(h)
    return h


def _gather_jnp(arr, nidx):
    B, S, K = nidx.shape
    out = jax.vmap(lambda a, i: a[i])(arr, nidx.reshape(B, S * K))
    return out.reshape(B, S, K, arr.shape[-1])


def _sa_jnp(x, pos, ratio, radius, layers, K=32):
    B, P, _ = pos.shape
    S = int(P * ratio)
    idx = jax.vmap(partial(_fps_jnp, S=S))(jax.lax.stop_gradient(pos))
    cent = jnp.take_along_axis(pos, idx[:, :, None], axis=1)
    d2 = jnp.sum((cent[:, :, None, :] - pos[:, None, :, :]) ** 2, axis=-1)
    neg_top, nidx = jax.lax.top_k(-d2, K)
    mask = (-neg_top) <= radius ** 2
    nb_pos = _gather_jnp(pos, nidx)
    nb_x = _gather_jnp(x, nidx)
    rel = nb_pos - cent[:, :, None, :]
    feats = jnp.concatenate([nb_x, rel], axis=-1)
    C = feats.shape[-1]
    h = _mlp_bn_jnp(feats.reshape(-1, C), mask.reshape(-1), layers)
    h = h.reshape(B, S, K, -1)
    h = jnp.where(mask[..., None], h, -jnp.inf)
    return jnp.max(h, axis=2), cent


def _ga_head_kernel(h_ref, w1_ref, b1_ref, g1_ref, be1_ref,
                    w2_ref, b2_ref, g2_ref, be2_ref,
                    l1w_ref, l1b_ref, l2w_ref, l2b_ref, l3w_ref, l3b_ref,
                    out_ref, *, B, S):
    h = h_ref[...]
    for (w, b, g, be) in ((w1_ref, b1_ref, g1_ref, be1_ref),
                          (w2_ref, b2_ref, g2_ref, be2_ref)):
        h = jnp.dot(h, w[...], preferred_element_type=jnp.float32) + b[...]
        mu = jnp.mean(h, axis=0)
        var = jnp.mean((h - mu) ** 2, axis=0)
        h = (h - mu) / jnp.sqrt(var + 1e-5) * g[...] + be[...]
        h = jnp.maximum(h, 0.0)
    h = jnp.max(h.reshape(B, S, h.shape[-1]), axis=1)
    h = jnp.maximum(jnp.dot(h, l1w_ref[...], preferred_element_type=jnp.float32) + l1b_ref[...], 0.0)
    h = jnp.maximum(jnp.dot(h, l2w_ref[...], preferred_element_type=jnp.float32) + l2b_ref[...], 0.0)
    out_ref[...] = jnp.dot(h, l3w_ref[...], preferred_element_type=jnp.float32) + l3b_ref[...]


def _ga_head(x, pos, params):
    B, S, _ = x.shape
    h = jnp.concatenate([x, pos], axis=-1).reshape(B * S, -1)
    (w1, b1, g1, be1), (w2, b2, g2, be2) = params["ga"]
    l1w, l1b = params["lin1"]
    l2w, l2b = params["lin2"]
    l3w, l3b = params["lin3"]
    out = pl.pallas_call(
        partial(_ga_head_kernel, B=B, S=S),
        out_shape=jax.ShapeDtypeStruct((B, l3w.shape[1]), jnp.float32),
    )(h, w1, b1, g1, be1, w2, b2, g2, be2, l1w, l1b, l2w, l2b, l3w, l3b)
    return out


def kernel(x, pos, batch, params):
    B = batch.shape[0] // 2048
    P = x.shape[0] // B
    x = x.reshape(B, P, -1)
    pos = pos.reshape(B, P, 3)
    x, pos = _sa_jnp(x, pos, 0.5, 0.2, params["sa1"])
    x, pos = _sa_jnp(x, pos, 0.25, 0.4, params["sa2"])
    return _ga_head(x, pos, params)


# speedup vs baseline: 1.4542x; 1.4260x over previous
import jax
import jax.numpy as jnp
import numpy as np
from functools import partial
from jax.experimental import pallas as pl
from jax.experimental.pallas import tpu as pltpu


def _fps_kernel(px_ref, py_ref, pz_ref, cx_ref, cy_ref, cz_ref, *, S, P):
    px = px_ref[...]
    py = py_ref[...]
    pz = pz_ref[...]
    B = px.shape[0]
    iota_p = jax.lax.broadcasted_iota(jnp.int32, (B, P), 1)
    iota_s = jax.lax.broadcasted_iota(jnp.int32, (B, S), 1)
    lx0 = px[:, 0:1]
    ly0 = py[:, 0:1]
    lz0 = pz[:, 0:1]
    cx_ref[...] = jnp.where(iota_s == 0, lx0, 0.0)
    cy_ref[...] = jnp.where(iota_s == 0, ly0, 0.0)
    cz_ref[...] = jnp.where(iota_s == 0, lz0, 0.0)

    def body(i, state):
        dist, lx, ly, lz = state
        dx = px - lx
        dy = py - ly
        dz = pz - lz
        d = dx * dx + dy * dy
        d = d + dz * dz
        dist = jnp.minimum(dist, d)
        m = jnp.max(dist, axis=1, keepdims=True)
        idx = jnp.min(jnp.where(dist == m, iota_p, P), axis=1, keepdims=True)
        sel = iota_p == idx
        ninf = jnp.float32(-jnp.inf)
        lx = jnp.max(jnp.where(sel, px, ninf), axis=1, keepdims=True)
        ly = jnp.max(jnp.where(sel, py, ninf), axis=1, keepdims=True)
        lz = jnp.max(jnp.where(sel, pz, ninf), axis=1, keepdims=True)
        oh = iota_s == i
        cx_ref[...] = jnp.where(oh, lx, cx_ref[...])
        cy_ref[...] = jnp.where(oh, ly, cy_ref[...])
        cz_ref[...] = jnp.where(oh, lz, cz_ref[...])
        return dist, lx, ly, lz

    dist0 = jnp.full((B, P), jnp.inf, dtype=jnp.float32)
    jax.lax.fori_loop(1, S, body, (dist0, lx0, ly0, lz0))


def _fps_pallas(pos, S):
    B, P, _ = pos.shape
    px = pos[:, :, 0]
    py = pos[:, :, 1]
    pz = pos[:, :, 2]
    out = jax.ShapeDtypeStruct((B, S), jnp.float32)
    cx, cy, cz = pl.pallas_call(
        partial(_fps_kernel, S=S, P=P),
        out_shape=(out, out, out),
    )(px, py, pz)
    return jnp.stack([cx, cy, cz], axis=-1)


def _mlp_bn_jnp(h, mask, layers):
    for (W, b, g, beta) in layers:
        h = h @ W + b
        if mask is None:
            mu = jnp.mean(h, axis=0)
            var = jnp.mean((h - mu) ** 2, axis=0)
        else:
            m = mask[:, None].astype(h.dtype)
            cnt = jnp.maximum(jnp.sum(m), 1.0)
            mu = jnp.sum(h * m, axis=0) / cnt
            var = jnp.sum(((h - mu) ** 2) * m, axis=0) / cnt
        h = (h - mu) / jnp.sqrt(var + 1e-5) * g + beta
        h = jax.nn.relu(h)
    return h


def _gather_jnp(arr, nidx):
    B, S, K = nidx.shape
    out = jax.vmap(lambda a, i: a[i])(arr, nidx.reshape(B, S * K))
    return out.reshape(B, S, K, arr.shape[-1])


def _sa_jnp(x, pos, ratio, radius, layers, K=32):
    B, P, _ = pos.shape
    S = int(P * ratio)
    cent = _fps_pallas(pos, S)
    d2 = jnp.sum((cent[:, :, None, :] - pos[:, None, :, :]) ** 2, axis=-1)
    neg_top, nidx = jax.lax.top_k(-d2, K)
    mask = (-neg_top) <= radius ** 2
    nb_pos = _gather_jnp(pos, nidx)
    nb_x = _gather_jnp(x, nidx)
    rel = nb_pos - cent[:, :, None, :]
    feats = jnp.concatenate([nb_x, rel], axis=-1)
    C = feats.shape[-1]
    h = _mlp_bn_jnp(feats.reshape(-1, C), mask.reshape(-1), layers)
    h = h.reshape(B, S, K, -1)
    h = jnp.where(mask[..., None], h, -jnp.inf)
    return jnp.max(h, axis=2), cent


def _ga_head_kernel(h_ref, w1_ref, b1_ref, g1_ref, be1_ref,
                    w2_ref, b2_ref, g2_ref, be2_ref,
                    l1w_ref, l1b_ref, l2w_ref, l2b_ref, l3w_ref, l3b_ref,
                    out_ref, *, B, S):
    h = h_ref[...]
    for (w, b, g, be) in ((w1_ref, b1_ref, g1_ref, be1_ref),
                          (w2_ref, b2_ref, g2_ref, be2_ref)):
        h = jnp.dot(h, w[...], preferred_element_type=jnp.float32) + b[...]
        mu = jnp.mean(h, axis=0)
        var = jnp.mean((h - mu) ** 2, axis=0)
        h = (h - mu) / jnp.sqrt(var + 1e-5) * g[...] + be[...]
        h = jnp.maximum(h, 0.0)
    h = jnp.max(h.reshape(B, S, h.shape[-1]), axis=1)
    h = jnp.maximum(jnp.dot(h, l1w_ref[...], preferred_element_type=jnp.float32) + l1b_ref[...], 0.0)
    h = jnp.maximum(jnp.dot(h, l2w_ref[...], preferred_element_type=jnp.float32) + l2b_ref[...], 0.0)
    out_ref[...] = jnp.dot(h, l3w_ref[...], preferred_element_type=jnp.float32) + l3b_ref[...]


def _ga_head(x, pos, params):
    B, S, _ = x.shape
    h = jnp.concatenate([x, pos], axis=-1).reshape(B * S, -1)
    (w1, b1, g1, be1), (w2, b2, g2, be2) = params["ga"]
    l1w, l1b = params["lin1"]
    l2w, l2b = params["lin2"]
    l3w, l3b = params["lin3"]
    out = pl.pallas_call(
        partial(_ga_head_kernel, B=B, S=S),
        out_shape=jax.ShapeDtypeStruct((B, l3w.shape[1]), jnp.float32),
    )(h, w1, b1, g1, be1, w2, b2, g2, be2, l1w, l1b, l2w, l2b, l3w, l3b)
    return out


def kernel(x, pos, batch, params):
    B = batch.shape[0] // 2048
    P = x.shape[0] // B
    x = x.reshape(B, P, -1)
    pos = pos.reshape(B, P, 3)
    x, pos = _sa_jnp(x, pos, 0.5, 0.2, params["sa1"])
    x, pos = _sa_jnp(x, pos, 0.25, 0.4, params["sa2"])
    return _ga_head(x, pos, params)


# speedup vs baseline: 5.9232x; 4.0731x over previous
import jax
import jax.numpy as jnp
import numpy as np
from functools import partial
from jax.experimental import pallas as pl
from jax.experimental.pallas import tpu as pltpu


def _fps_kernel(px_ref, py_ref, pz_ref, cx_ref, cy_ref, cz_ref, *, S, P):
    px = px_ref[...]
    py = py_ref[...]
    pz = pz_ref[...]
    B = px.shape[0]
    iota_p = jax.lax.broadcasted_iota(jnp.int32, (B, P), 1)
    iota_s = jax.lax.broadcasted_iota(jnp.int32, (B, S), 1)
    lx0 = px[:, 0:1]
    ly0 = py[:, 0:1]
    lz0 = pz[:, 0:1]
    cx_ref[...] = jnp.where(iota_s == 0, lx0, 0.0)
    cy_ref[...] = jnp.where(iota_s == 0, ly0, 0.0)
    cz_ref[...] = jnp.where(iota_s == 0, lz0, 0.0)

    def body(i, state):
        dist, lx, ly, lz = state
        dx = px - lx
        dy = py - ly
        dz = pz - lz
        d = dx * dx + dy * dy
        d = d + dz * dz
        dist = jnp.minimum(dist, d)
        m = jnp.max(dist, axis=1, keepdims=True)
        idx = jnp.min(jnp.where(dist == m, iota_p, P), axis=1, keepdims=True)
        sel = iota_p == idx
        ninf = jnp.float32(-jnp.inf)
        lx = jnp.max(jnp.where(sel, px, ninf), axis=1, keepdims=True)
        ly = jnp.max(jnp.where(sel, py, ninf), axis=1, keepdims=True)
        lz = jnp.max(jnp.where(sel, pz, ninf), axis=1, keepdims=True)
        oh = iota_s == i
        cx_ref[...] = jnp.where(oh, lx, cx_ref[...])
        cy_ref[...] = jnp.where(oh, ly, cy_ref[...])
        cz_ref[...] = jnp.where(oh, lz, cz_ref[...])
        return dist, lx, ly, lz

    dist0 = jnp.full((B, P), jnp.inf, dtype=jnp.float32)
    jax.lax.fori_loop(1, S, body, (dist0, lx0, ly0, lz0))


def _fps_pallas(pos, S):
    B, P, _ = pos.shape
    px = pos[:, :, 0]
    py = pos[:, :, 1]
    pz = pos[:, :, 2]
    out = jax.ShapeDtypeStruct((B, S), jnp.float32)
    cx, cy, cz = pl.pallas_call(
        partial(_fps_kernel, S=S, P=P),
        out_shape=(out, out, out),
    )(px, py, pz)
    return jnp.stack([cx, cy, cz], axis=-1)


def _knn_kernel(px_ref, py_ref, pz_ref, cx_ref, cy_ref, cz_ref,
                nidx_ref, maskf_ref, *, P, T, K, r2):
    b = pl.program_id(0)
    px = px_ref[pl.ds(b, 1), :]
    py = py_ref[pl.ds(b, 1), :]
    pz = pz_ref[pl.ds(b, 1), :]
    cx = cx_ref[...]
    cy = cy_ref[...]
    cz = cz_ref[...]
    dx = cx - px
    dy = cy - py
    dz = cz - pz
    D = dx * dx + dy * dy + dz * dz
    iota_p = jax.lax.broadcasted_iota(jnp.int32, (T, P), 1)
    idx_cols = []
    msk_cols = []
    for _ in range(K):
        m = jnp.min(D, axis=1, keepdims=True)
        idx = jnp.min(jnp.where(D == m, iota_p, P), axis=1, keepdims=True)
        D = jnp.where(iota_p == idx, jnp.inf, D)
        idx_cols.append(idx + b * P)
        msk_cols.append((m <= r2).astype(jnp.float32))
    nidx_ref[...] = jnp.concatenate(idx_cols, axis=1)[None, None]
    maskf_ref[...] = jnp.concatenate(msk_cols, axis=1)[None, None]


def _knn_pallas(pos_bp, cent, radius, K=32, T=256):
    px, py, pz = pos_bp
    B, P = px.shape
    S = cent.shape[1]
    T = min(T, S)
    cxT = cent[:, :, 0].reshape(B * S, 1)
    cyT = cent[:, :, 1].reshape(B * S, 1)
    czT = cent[:, :, 2].reshape(B * S, 1)
    nt = S // T
    grid = (B, nt)
    pos_spec = pl.BlockSpec((B, P), lambda b, t: (0, 0))
    cent_spec = pl.BlockSpec((T, 1), lambda b, t: (b * nt + t, 0))
    out_spec = pl.BlockSpec((1, 1, T, K), lambda b, t: (b, t, 0, 0))
    nidx, maskf = pl.pallas_call(
        partial(_knn_kernel, P=P, T=T, K=K, r2=radius * radius),
        grid=grid,
        in_specs=[pos_spec, pos_spec, pos_spec, cent_spec, cent_spec, cent_spec],
        out_specs=(out_spec, out_spec),
        out_shape=(jax.ShapeDtypeStruct((B, nt, T, K), jnp.int32),
                   jax.ShapeDtypeStruct((B, nt, T, K), jnp.float32)),
    )(px, py, pz, cxT, cyT, czT)
    return nidx.reshape(B, S, K), maskf.reshape(B, S, K)


def _mlp_bn_jnp(h, mask, layers):
    for (W, b, g, beta) in layers:
        h = h @ W + b
        if mask is None:
            mu = jnp.mean(h, axis=0)
            var = jnp.mean((h - mu) ** 2, axis=0)
        else:
            m = mask[:, None].astype(h.dtype)
            cnt = jnp.maximum(jnp.sum(m), 1.0)
            mu = jnp.sum(h * m, axis=0) / cnt
            var = jnp.sum(((h - mu) ** 2) * m, axis=0) / cnt
        h = (h - mu) / jnp.sqrt(var + 1e-5) * g + beta
        h = jax.nn.relu(h)
    return h


def _sa_jnp(x, pos, ratio, radius, layers, K=32):
    B, P, _ = pos.shape
    S = int(P * ratio)
    cent = _fps_pallas(pos, S)
    pos_bp = (pos[:, :, 0], pos[:, :, 1], pos[:, :, 2])
    nidx, maskf = _knn_pallas(pos_bp, cent, radius, K=K)
    mask = maskf > 0.5
    flat = nidx.reshape(-1)
    nb_pos = pos.reshape(B * P, 3)[flat].reshape(B, S, K, 3)
    nb_x = x.reshape(B * P, -1)[flat].reshape(B, S, K, -1)
    rel = nb_pos - cent[:, :, None, :]
    feats = jnp.concatenate([nb_x, rel], axis=-1)
    C = feats.shape[-1]
    h = _mlp_bn_jnp(feats.reshape(-1, C), mask.reshape(-1), layers)
    h = h.reshape(B, S, K, -1)
    h = jnp.where(mask[..., None], h, -jnp.inf)
    return jnp.max(h, axis=2), cent


def _ga_head_kernel(h_ref, w1_ref, b1_ref, g1_ref, be1_ref,
                    w2_ref, b2_ref, g2_ref, be2_ref,
                    l1w_ref, l1b_ref, l2w_ref, l2b_ref, l3w_ref, l3b_ref,
                    out_ref, *, B, S):
    h = h_ref[...]
    for (w, b, g, be) in ((w1_ref, b1_ref, g1_ref, be1_ref),
                          (w2_ref, b2_ref, g2_ref, be2_ref)):
        h = jnp.dot(h, w[...], preferred_element_type=jnp.float32) + b[...]
        mu = jnp.mean(h, axis=0)
        var = jnp.mean((h - mu) ** 2, axis=0)
        h = (h - mu) / jnp.sqrt(var + 1e-5) * g[...] + be[...]
        h = jnp.maximum(h, 0.0)
    h = jnp.max(h.reshape(B, S, h.shape[-1]), axis=1)
    h = jnp.maximum(jnp.dot(h, l1w_ref[...], preferred_element_type=jnp.float32) + l1b_ref[...], 0.0)
    h = jnp.maximum(jnp.dot(h, l2w_ref[...], preferred_element_type=jnp.float32) + l2b_ref[...], 0.0)
    out_ref[...] = jnp.dot(h, l3w_ref[...], preferred_element_type=jnp.float32) + l3b_ref[...]


def _ga_head(x, pos, params):
    B, S, _ = x.shape
    h = jnp.concatenate([x, pos], axis=-1).reshape(B * S, -1)
    (w1, b1, g1, be1), (w2, b2, g2, be2) = params["ga"]
    l1w, l1b = params["lin1"]
    l2w, l2b = params["lin2"]
    l3w, l3b = params["lin3"]
    out = pl.pallas_call(
        partial(_ga_head_kernel, B=B, S=S),
        out_shape=jax.ShapeDtypeStruct((B, l3w.shape[1]), jnp.float32),
    )(h, w1, b1, g1, be1, w2, b2, g2, be2, l1w, l1b, l2w, l2b, l3w, l3b)
    return out


def kernel(x, pos, batch, params):
    B = batch.shape[0] // 2048
    P = x.shape[0] // B
    x = x.reshape(B, P, -1)
    pos = pos.reshape(B, P, 3)
    x, pos = _sa_jnp(x, pos, 0.5, 0.2, params["sa1"])
    x, pos = _sa_jnp(x, pos, 0.25, 0.4, params["sa2"])
    return _ga_head(x, pos, params)
